# baseline (speedup 1.0000x reference)
import functools

import jax
import jax.numpy as jnp
from jax import lax
from jax.experimental import pallas as pl
from jax.experimental.pallas import tpu as pltpu
from jax.experimental.pallas import tpu_sc as plsc

_TM = 3072
_KCHUNKS = 2


def _vq_tc_body(x_ref, w_ref, b_ref, book_ref, z_ref, idx_ref, loss_ref, aug_ref):
    i = pl.program_id(0)

    @pl.when(i == 0)
    def _():
        book = book_ref[...]
        aug_ref[...] = 0.5 * jnp.sum(book * book, axis=1, keepdims=True)
        loss_ref[...] = jnp.zeros_like(loss_ref)

    z = lax.dot_general(
        x_ref[...], w_ref[...],
        dimension_numbers=(((1,), (1,)), ((), ())),
        preferred_element_type=jnp.float32,
    ) + b_ref[...]
    kc = book_ref.shape[0] // _KCHUNKS
    ms, ids = [], []
    for j in range(_KCHUNKS):
        sj = lax.dot_general(
            book_ref[pl.ds(j * kc, kc), :], z,
            dimension_numbers=(((1,), (1,)), ((), ())),
            preferred_element_type=jnp.float32,
        ) - aug_ref[pl.ds(j * kc, kc), :]
        ms.append(jnp.max(sj, axis=0, keepdims=True))
        ids.append(
            jnp.argmax(sj, axis=0).astype(jnp.int32).reshape(1, -1) + j * kc
        )
    smaxT, idxT = ms[0], ids[0]
    for j in range(1, _KCHUNKS):
        better = smaxT >= ms[j]
        idxT = jnp.where(better, idxT, ids[j])
        smaxT = jnp.where(better, smaxT, ms[j])
    z_ref[...] = z
    idx_ref[...] = idxT
    part = jnp.sum(z * z, axis=(0, 1), keepdims=True) - 2.0 * jnp.sum(
        smaxT, axis=(0, 1), keepdims=True
    )
    loss_ref[...] += part


def _vq_tc(xf, w, b2, book):
    m, d = xf.shape
    k, c = book.shape
    return pl.pallas_call(
        _vq_tc_body,
        grid=(m // _TM,),
        in_specs=[
            pl.BlockSpec((_TM, d), lambda i: (i, 0)),
            pl.BlockSpec((c, d), lambda i: (0, 0)),
            pl.BlockSpec((1, c), lambda i: (0, 0)),
            pl.BlockSpec((k, c), lambda i: (0, 0)),
        ],
        out_specs=[
            pl.BlockSpec((_TM, c), lambda i: (i, 0)),
            pl.BlockSpec((1, _TM), lambda i: (0, i)),
            pl.BlockSpec((1, 1), lambda i: (0, 0)),
        ],
        out_shape=[
            jax.ShapeDtypeStruct((m, c), jnp.float32),
            jax.ShapeDtypeStruct((1, m), jnp.int32),
            jax.ShapeDtypeStruct((1, 1), jnp.float32),
        ],
        scratch_shapes=[pltpu.VMEM((k, 1), jnp.float32)],
    )(xf, w, b2, book)


def _sc_gather(book, idx_flat):
    info = plsc.get_sparse_core_info()
    nc, ns = info.num_cores, info.num_subcores
    nw = nc * ns
    m = idx_flat.shape[0]
    kk, c = book.shape
    bpw = m // nw
    chunks = [(o, min(128, bpw - o)) for o in range(0, bpw, 128)]
    mesh = plsc.VectorSubcoreMesh(core_axis_name="c", subcore_axis_name="s")

    lanes = info.num_lanes
    ngrp = bpw // lanes

    @functools.partial(
        pl.kernel,
        out_type=jax.ShapeDtypeStruct((m * c,), jnp.float32),
        mesh=mesh,
        compiler_params=pltpu.CompilerParams(needs_layout_passes=False),
        scratch_types=[
            pltpu.VMEM((kk * c,), jnp.float32),
            pltpu.VMEM((bpw,), jnp.int32),
            pltpu.VMEM((bpw * c,), jnp.float32),
        ],
    )
    def gk(table_hbm, idx_hbm, out_hbm, table_v, idx_v, rows_v):
        wid = lax.axis_index("s") * nc + lax.axis_index("c")
        base = wid * bpw
        pltpu.sync_copy(table_hbm, table_v)
        pltpu.sync_copy(idx_hbm.at[pl.ds(base, bpw)], idx_v)
        lane = lax.iota(jnp.int32, lanes)

        def body(g, carry):
            v16 = idx_v[pl.ds(g * lanes, lanes)]
            for j in range(lanes):
                spl = lax.gather(
                    v16,
                    jnp.full((lanes, 1), j, jnp.int32),
                    lax.GatherDimensionNumbers(
                        offset_dims=(),
                        collapsed_slice_dims=(0,),
                        start_index_map=(0,),
                    ),
                    slice_sizes=(1,),
                    mode=lax.GatherScatterMode.PROMISE_IN_BOUNDS,
                )
                flat = spl * c
                r = g * lanes + j
                for h in range(c // lanes):
                    vals = plsc.load_gather(table_v, [flat + (lane + h * lanes)])
                    rows_v[pl.ds(r * c + h * lanes, lanes)] = vals
            return carry

        lax.fori_loop(0, ngrp, body, 0)
        pltpu.sync_copy(rows_v, out_hbm.at[pl.ds(base * c, bpw * c)])

    return gk(book.reshape(kk * c), idx_flat)


def kernel(x, codebook, W_down, b_down):
    b, t, dm = x.shape
    book = codebook[0]
    k, c = book.shape
    b2 = b_down.reshape(1, -1)
    bh = b // 2
    halves = []
    for h in range(2):
        xh = x[h * bh : (h + 1) * bh].reshape(bh * t, dm)
        z_f, idx_f, loss_sum = _vq_tc(xh, W_down, b2, book)
        hard_flat = _sc_gather(book, idx_f.reshape(bh * t))
        halves.append((z_f, idx_f, loss_sum, hard_flat))
    z = jnp.concatenate([hv[0].reshape(bh, t, c) for hv in halves], axis=0)
    code_indices = jnp.concatenate(
        [hv[1].reshape(bh, t) for hv in halves], axis=0
    )
    hard_codes_st = jnp.concatenate(
        [hv[3].reshape(bh, t, c) for hv in halves], axis=0
    )
    loss = (halves[0][2][0, 0] + halves[1][2][0, 0]) / (b * t * c)
    return (z, code_indices, hard_codes_st, loss, loss)


# speedup vs baseline: 1.6099x; 1.6099x over previous
import functools

import jax
import jax.numpy as jnp
from jax import lax
from jax.experimental import pallas as pl
from jax.experimental.pallas import tpu as pltpu
from jax.experimental.pallas import tpu_sc as plsc

_TM = 3072
_KCHUNKS = 2


def _vq_tc_body(x_ref, w_ref, b_ref, book_ref, z_ref, idx_ref, loss_ref, aug_ref):
    i = pl.program_id(0)

    @pl.when(i == 0)
    def _():
        book = book_ref[...]
        aug_ref[...] = 0.5 * jnp.sum(book * book, axis=1, keepdims=True)
        loss_ref[...] = jnp.zeros_like(loss_ref)

    z = lax.dot_general(
        x_ref[...], w_ref[...],
        dimension_numbers=(((1,), (1,)), ((), ())),
        preferred_element_type=jnp.float32,
    ) + b_ref[...]
    kc = book_ref.shape[0] // _KCHUNKS
    ms, ids = [], []
    for j in range(_KCHUNKS):
        sj = lax.dot_general(
            book_ref[pl.ds(j * kc, kc), :], z,
            dimension_numbers=(((1,), (1,)), ((), ())),
            preferred_element_type=jnp.float32,
        ) - aug_ref[pl.ds(j * kc, kc), :]
        ms.append(jnp.max(sj, axis=0, keepdims=True))
        ids.append(
            jnp.argmax(sj, axis=0).astype(jnp.int32).reshape(1, -1) + j * kc
        )
    smaxT, idxT = ms[0], ids[0]
    for j in range(1, _KCHUNKS):
        better = smaxT >= ms[j]
        idxT = jnp.where(better, idxT, ids[j])
        smaxT = jnp.where(better, smaxT, ms[j])
    z_ref[...] = z
    idx_ref[...] = idxT
    part = jnp.sum(z * z, axis=(0, 1), keepdims=True) - 2.0 * jnp.sum(
        smaxT, axis=(0, 1), keepdims=True
    )
    loss_ref[...] += part


def _vq_tc(xf, w, b2, book):
    m, d = xf.shape
    k, c = book.shape
    return pl.pallas_call(
        _vq_tc_body,
        grid=(m // _TM,),
        in_specs=[
            pl.BlockSpec((_TM, d), lambda i: (i, 0)),
            pl.BlockSpec((c, d), lambda i: (0, 0)),
            pl.BlockSpec((1, c), lambda i: (0, 0)),
            pl.BlockSpec((k, c), lambda i: (0, 0)),
        ],
        out_specs=[
            pl.BlockSpec((_TM, c), lambda i: (i, 0)),
            pl.BlockSpec((1, _TM), lambda i: (0, i)),
            pl.BlockSpec((1, 1), lambda i: (0, 0)),
        ],
        out_shape=[
            jax.ShapeDtypeStruct((m, c), jnp.float32),
            jax.ShapeDtypeStruct((1, m), jnp.int32),
            jax.ShapeDtypeStruct((1, 1), jnp.float32),
        ],
        scratch_shapes=[pltpu.VMEM((k, 1), jnp.float32)],
    )(xf, w, b2, book)


def _sc_gather(book, idx_flat):
    info = plsc.get_sparse_core_info()
    nc, ns = info.num_cores, info.num_subcores
    nw = nc * ns
    m = idx_flat.shape[0]
    kk, c = book.shape
    bpw = m // nw
    chunks = [(o, min(128, bpw - o)) for o in range(0, bpw, 128)]
    mesh = plsc.VectorSubcoreMesh(core_axis_name="c", subcore_axis_name="s")

    lanes = info.num_lanes
    ngrp = bpw // lanes

    @functools.partial(
        pl.kernel,
        out_type=jax.ShapeDtypeStruct((m * c,), jnp.float32),
        mesh=mesh,
        compiler_params=pltpu.CompilerParams(needs_layout_passes=False),
        scratch_types=[
            pltpu.VMEM((kk * c,), jnp.float32),
            pltpu.VMEM((bpw,), jnp.int32),
            pltpu.VMEM((bpw * c,), jnp.float32),
        ],
    )
    def gk(table_hbm, idx_hbm, out_hbm, table_v, idx_v, rows_v):
        wid = lax.axis_index("s") * nc + lax.axis_index("c")
        base = wid * bpw
        pltpu.sync_copy(table_hbm, table_v)
        pltpu.sync_copy(idx_hbm.at[pl.ds(base, bpw)], idx_v)
        lane = lax.iota(jnp.int32, lanes)

        @functools.partial(plsc.parallel_loop(0, ngrp, unroll=2))
        def _loop(g):
            v16 = idx_v[pl.ds(g * lanes, lanes)]
            for j in range(lanes):
                spl = lax.gather(
                    v16,
                    jnp.full((lanes, 1), j, jnp.int32),
                    lax.GatherDimensionNumbers(
                        offset_dims=(),
                        collapsed_slice_dims=(0,),
                        start_index_map=(0,),
                    ),
                    slice_sizes=(1,),
                    mode=lax.GatherScatterMode.PROMISE_IN_BOUNDS,
                )
                flat = spl * c
                r = g * lanes + j
                for h in range(c // lanes):
                    vals = plsc.load_gather(table_v, [flat + (lane + h * lanes)])
                    rows_v[pl.ds(r * c + h * lanes, lanes)] = vals

        pltpu.sync_copy(rows_v, out_hbm.at[pl.ds(base * c, bpw * c)])

    return gk(book.reshape(kk * c), idx_flat)


def kernel(x, codebook, W_down, b_down):
    b, t, dm = x.shape
    book = codebook[0]
    k, c = book.shape
    xf = x.reshape(b * t, dm)
    z_f, idx_f, loss_sum = _vq_tc(xf, W_down, b_down.reshape(1, -1), book)
    hard_flat = _sc_gather(book, idx_f.reshape(b * t))
    z = z_f.reshape(b, t, c)
    code_indices = idx_f.reshape(b, t)
    hard_codes_st = hard_flat.reshape(b, t, c)
    loss = loss_sum[0, 0] / (b * t * c)
    return (z, code_indices, hard_codes_st, loss, loss)
